# baseline (speedup 1.0000x reference)
import jax
import jax.numpy as jnp
from jax import lax
from jax.experimental import pallas as pl
from jax.experimental.pallas import tpu as pltpu
from jax.experimental.pallas import tpu_sc as plsc

_N = 10000
_E = 160000
_H = 10

_NC = 2
_NS = 16
_NW = _NC * _NS
_CHUNK = 640
_IDXG = _CHUNK // 128
_KCH = 8
_EPW = _CHUNK * _KCH
_EPAD = _EPW * _NW
_RPS = 624
_RTAIL = _N - _NS * _RPS


def _mm_body(x_ref, w_ref, o_ref):
    o_ref[...] = jnp.dot(x_ref[...], w_ref[...],
                         preferred_element_type=jnp.float32)


def _big_matmul(x, w, bm):
    m, k = x.shape
    n = w.shape[1]
    return pl.pallas_call(
        _mm_body,
        grid=(m // bm,),
        in_specs=[pl.BlockSpec((bm, k), lambda i: (i, 0)),
                  pl.BlockSpec((k, n), lambda i: (0, 0))],
        out_specs=pl.BlockSpec((bm, n), lambda i: (i, 0)),
        out_shape=jax.ShapeDtypeStruct((m, n), jnp.float32),
    )(x, w)


def _make_spmm(d, cols):
    mesh = plsc.VectorSubcoreMesh(core_axis_name="c", subcore_axis_name="s")

    def body(table, src2, dst2, wvec, zrows, out,
             src_v, dst_v, w_v, rows_v, acc, gsems, ssems):
        c = lax.axis_index("c")
        s = lax.axis_index("s")
        wid = s * _NC + c
        r0 = s * _RPS

        pltpu.sync_copy(zrows.at[pl.ds(r0, _RPS)], acc.at[pl.ds(r0, _RPS)])

        @pl.when(s == 0)
        def _ztail():
            pltpu.sync_copy(zrows.at[pl.ds(_NS * _RPS, _RTAIL)],
                            acc.at[pl.ds(_NS * _RPS, _RTAIL)])

        plsc.subcore_barrier()

        def fetch(k, b):
            base = wid * _EPW + k * _CHUNK
            rb = wid * (_EPW // 128) + k * _IDXG
            pltpu.sync_copy(src2.at[pl.ds(rb, _IDXG)], src_v.at[b])
            pltpu.sync_copy(dst2.at[pl.ds(rb, _IDXG)], dst_v.at[b])
            pltpu.sync_copy(wvec.at[pl.ds(base, _CHUNK)], w_v.at[b])
            for j in range(_IDXG):
                pltpu.async_copy(table.at[src_v.at[b, j]],
                                 rows_v.at[b, pl.ds(j * 128, 128)],
                                 gsems.at[b])

        def scatter_descs(b):
            return [pltpu.make_async_copy(rows_v.at[b, pl.ds(j * 128, 128)],
                                          acc.at[dst_v.at[b, j]],
                                          ssems.at[b])
                    for j in range(_IDXG)]

        def process(k, b):
            for j in range(_IDXG):
                pltpu.make_async_copy(table.at[src_v.at[b, j]],
                                      rows_v.at[b, pl.ds(j * 128, 128)],
                                      gsems.at[b]).wait()

            @pl.loop(0, _CHUNK // 16)
            def _grp(g):
                w16 = w_v[b, pl.ds(g * 16, 16)]
                rid = g * 16 + lax.iota(jnp.int32, 16)
                for dcol in cols:
                    cidx = jnp.full((16,), dcol, jnp.int32)
                    v = plsc.load_gather(rows_v.at[b], [rid, cidx])
                    plsc.store_scatter(rows_v.at[b], [rid, cidx], v * w16)

            for dsc in scatter_descs(b):
                dsc.start(add=True)

        fetch(0, 0)
        fetch(1, 1)

        @pl.loop(0, _KCH // 2)
        def _pair(i):
            k = i * 2
            process(k, 0)
            process(k + 1, 1)

            @pl.when(k + 2 < _KCH)
            def _fetch_a():
                for dsc in scatter_descs(0):
                    dsc.wait()
                fetch(k + 2, 0)

            @pl.when(k + 3 < _KCH)
            def _fetch_b():
                for dsc in scatter_descs(1):
                    dsc.wait()
                fetch(k + 3, 1)

        for b in (0, 1):
            for dsc in scatter_descs(b):
                dsc.wait()

        plsc.subcore_barrier()
        pltpu.sync_copy(acc.at[pl.ds(r0, _RPS)],
                        out.at[c, pl.ds(r0, _RPS)])

        @pl.when(s == 0)
        def _otail():
            pltpu.sync_copy(acc.at[pl.ds(_NS * _RPS, _RTAIL)],
                            out.at[c, pl.ds(_NS * _RPS, _RTAIL)])

    return pl.kernel(
        body,
        out_type=jax.ShapeDtypeStruct((_NC, _N, d), jnp.float32),
        mesh=mesh,
        compiler_params=pltpu.CompilerParams(needs_layout_passes=False,
                                             use_tc_tiling_on_sc=False),
        scratch_types=[
            pltpu.VMEM((2, _IDXG, 128), jnp.int32),
            pltpu.VMEM((2, _IDXG, 128), jnp.int32),
            pltpu.VMEM((2, _CHUNK), jnp.float32),
            pltpu.VMEM((2, _CHUNK, d), jnp.float32),
            pltpu.VMEM_SHARED((_N, d), jnp.float32),
            pltpu.SemaphoreType.DMA((2,)),
            pltpu.SemaphoreType.DMA((2,)),
        ],
    )


def _make_spmm2():
    mesh = plsc.VectorSubcoreMesh(core_axis_name="c", subcore_axis_name="s")
    rows_pw = _EPW // 128

    def body(tabf, src2, dst2, wvec, out, tab_v, src_v, dst_v, w_v,
             acc_v, sem):
        c = lax.axis_index("c")
        s = lax.axis_index("s")
        wid = s * _NC + c
        rb = wid * rows_pw

        cp = pltpu.make_async_copy(tabf, tab_v, sem)
        cp.start()
        pltpu.sync_copy(src2.at[pl.ds(rb, rows_pw)], src_v)
        pltpu.sync_copy(dst2.at[pl.ds(rb, rows_pw)], dst_v)
        pltpu.sync_copy(wvec.at[pl.ds(wid * _EPW, _EPW)], w_v)

        zero16 = jnp.zeros((16,), jnp.float32)

        @pl.loop(0, 2 * _N // 16)
        def _z(i):
            acc_v[pl.ds(i * 16, 16)] = zero16

        cp.wait()

        @pl.loop(0, _EPW // 16)
        def _grp(g):
            row = g // 8
            off = (g % 8) * 16
            src16 = src_v[row, pl.ds(off, 16)]
            dst16 = dst_v[row, pl.ds(off, 16)]
            w16 = w_v[pl.ds(g * 16, 16)]
            i0 = src16 * 2
            j0 = dst16 * 2
            v0 = plsc.load_gather(tab_v, [i0]) * w16
            v1 = plsc.load_gather(tab_v, [i0 + 1]) * w16
            plsc.addupdate_scatter(acc_v, [j0], v0)
            plsc.addupdate_scatter(acc_v, [j0 + 1], v1)

        pltpu.sync_copy(acc_v, out.at[wid])

    return pl.kernel(
        body,
        out_type=jax.ShapeDtypeStruct((_NW, 2 * _N), jnp.float32),
        mesh=mesh,
        compiler_params=pltpu.CompilerParams(needs_layout_passes=False,
                                             use_tc_tiling_on_sc=False),
        scratch_types=[
            pltpu.VMEM((2 * _N,), jnp.float32),
            pltpu.VMEM((rows_pw, 128), jnp.int32),
            pltpu.VMEM((rows_pw, 128), jnp.int32),
            pltpu.VMEM((_EPW,), jnp.float32),
            pltpu.VMEM((2 * _N,), jnp.float32),
            pltpu.SemaphoreType.DMA,
        ],
    )


def _mid_body(p_ref, w_ref, o_ref):
    h = jnp.maximum(p_ref[0] + p_ref[1], 0.0)
    o_ref[...] = jnp.dot(h, w_ref[...], preferred_element_type=jnp.float32)


def _mid_fuse(p, wfull):
    return pl.pallas_call(
        _mid_body,
        out_shape=jax.ShapeDtypeStruct((_N, 2), jnp.float32),
    )(p, wfull)


def _out_body(p_ref, o_ref):
    t = jnp.sum(p_ref[...], axis=0)
    o_ref[...] = jax.nn.sigmoid(t)


def _out_fuse(p):
    return pl.pallas_call(
        _out_body,
        out_shape=jax.ShapeDtypeStruct((2 * _N,), jnp.float32),
    )(p)


def kernel(features, edge_index, edge_weight, W1, W2, W3, W4):
    src = edge_index[0]
    dst = edge_index[1]

    wc = jnp.zeros((_N, 32), jnp.float32)
    wc = wc.at[:, 0:_H].set(W1)
    wc = wc.at[:, 16:16 + _H].set(W3)
    xw = _big_matmul(features, wc, 400)

    src_p = jnp.zeros((_EPAD,), jnp.int32).at[:_E].set(src)
    src_p = src_p.reshape(_EPAD // 128, 128)
    dst_p = jnp.zeros((_EPAD,), jnp.int32).at[:_E].set(dst)
    dst_p = dst_p.reshape(_EPAD // 128, 128)
    w_p = jnp.zeros((_EPAD,), jnp.float32).at[:_E].set(edge_weight)

    spmm1 = _make_spmm(32, tuple(range(_H)) + tuple(range(16, 16 + _H)))
    z32 = jnp.zeros((_N, 32), jnp.float32)
    p1 = spmm1(xw, src_p, dst_p, w_p, z32)

    wfull = jnp.zeros((32, 2), jnp.float32)
    wfull = wfull.at[0:_H, 0].set(W2[:, 0])
    wfull = wfull.at[16:16 + _H, 1].set(W4[:, 0])
    hw = _mid_fuse(p1, wfull)

    spmm2 = _make_spmm2()
    p2 = spmm2(hw.reshape(-1), src_p, dst_p, w_p)

    o = _out_fuse(p2).reshape(_N, 2)
    zt = jnp.zeros((2 * _N,), jnp.float32)
    pd = _make_spmm2()(zt, src_p, dst_p, w_p)
    o = o + pd[0, 0]
    return o[:, 0:1], o[:, 1:2]


# speedup vs baseline: 1.0662x; 1.0662x over previous
import jax
import jax.numpy as jnp
from jax import lax
from jax.experimental import pallas as pl
from jax.experimental.pallas import tpu as pltpu
from jax.experimental.pallas import tpu_sc as plsc

_N = 10000
_E = 160000
_H = 10

_NC = 2
_NS = 16
_NW = _NC * _NS
_CHUNK = 640
_IDXG = _CHUNK // 128
_KCH = 8
_EPW = _CHUNK * _KCH
_EPAD = _EPW * _NW
_RPS = 624
_RTAIL = _N - _NS * _RPS


def _mm_body(x_ref, w_ref, o_ref):
    o_ref[...] = jnp.dot(x_ref[...], w_ref[...],
                         preferred_element_type=jnp.float32)


def _big_matmul(x, w, bm):
    m, k = x.shape
    n = w.shape[1]
    return pl.pallas_call(
        _mm_body,
        grid=(m // bm,),
        in_specs=[pl.BlockSpec((bm, k), lambda i: (i, 0)),
                  pl.BlockSpec((k, n), lambda i: (0, 0))],
        out_specs=pl.BlockSpec((bm, n), lambda i: (i, 0)),
        out_shape=jax.ShapeDtypeStruct((m, n), jnp.float32),
    )(x, w)


def _make_spmm(d, cols):
    mesh = plsc.VectorSubcoreMesh(core_axis_name="c", subcore_axis_name="s")

    def body(table, src2, dst2, wvec, zrows, out,
             src_v, dst_v, w_v, rows_v, acc, gsems, ssems):
        c = lax.axis_index("c")
        s = lax.axis_index("s")
        wid = s * _NC + c
        r0 = s * _RPS

        pltpu.sync_copy(zrows.at[pl.ds(r0, _RPS)], acc.at[pl.ds(r0, _RPS)])

        @pl.when(s == 0)
        def _ztail():
            pltpu.sync_copy(zrows.at[pl.ds(_NS * _RPS, _RTAIL)],
                            acc.at[pl.ds(_NS * _RPS, _RTAIL)])

        plsc.subcore_barrier()

        def fetch(k, b):
            base = wid * _EPW + k * _CHUNK
            rb = wid * (_EPW // 128) + k * _IDXG
            pltpu.sync_copy(src2.at[pl.ds(rb, _IDXG)], src_v.at[b])
            pltpu.sync_copy(dst2.at[pl.ds(rb, _IDXG)], dst_v.at[b])
            pltpu.sync_copy(wvec.at[pl.ds(base, _CHUNK)], w_v.at[b])
            for j in range(_IDXG):
                pltpu.async_copy(table.at[src_v.at[b, j]],
                                 rows_v.at[b, pl.ds(j * 128, 128)],
                                 gsems.at[b])

        def scatter_descs(b):
            return [pltpu.make_async_copy(rows_v.at[b, pl.ds(j * 128, 128)],
                                          acc.at[dst_v.at[b, j]],
                                          ssems.at[b])
                    for j in range(_IDXG)]

        def process(k, b):
            for j in range(_IDXG):
                pltpu.make_async_copy(table.at[src_v.at[b, j]],
                                      rows_v.at[b, pl.ds(j * 128, 128)],
                                      gsems.at[b]).wait()

                @pl.loop(0, 128 // 16, unroll=2)
                def _grp(g0, j=j):
                    g = j * 8 + g0
                    w16 = w_v[b, pl.ds(g * 16, 16)]
                    rid = g * 16 + lax.iota(jnp.int32, 16)
                    for dcol in cols:
                        cidx = jnp.full((16,), dcol, jnp.int32)
                        v = plsc.load_gather(rows_v.at[b], [rid, cidx])
                        plsc.store_scatter(rows_v.at[b], [rid, cidx],
                                           v * w16)

                pltpu.make_async_copy(rows_v.at[b, pl.ds(j * 128, 128)],
                                      acc.at[dst_v.at[b, j]],
                                      ssems.at[b]).start(add=True)

        fetch(0, 0)
        fetch(1, 1)

        @pl.loop(0, _KCH // 2)
        def _pair(i):
            k = i * 2
            process(k, 0)
            process(k + 1, 1)

            @pl.when(k + 2 < _KCH)
            def _fetch_a():
                for dsc in scatter_descs(0):
                    dsc.wait()
                fetch(k + 2, 0)

            @pl.when(k + 3 < _KCH)
            def _fetch_b():
                for dsc in scatter_descs(1):
                    dsc.wait()
                fetch(k + 3, 1)

        for b in (0, 1):
            for dsc in scatter_descs(b):
                dsc.wait()

        plsc.subcore_barrier()
        pltpu.sync_copy(acc.at[pl.ds(r0, _RPS)],
                        out.at[c, pl.ds(r0, _RPS)])

        @pl.when(s == 0)
        def _otail():
            pltpu.sync_copy(acc.at[pl.ds(_NS * _RPS, _RTAIL)],
                            out.at[c, pl.ds(_NS * _RPS, _RTAIL)])

    return pl.kernel(
        body,
        out_type=jax.ShapeDtypeStruct((_NC, _N, d), jnp.float32),
        mesh=mesh,
        compiler_params=pltpu.CompilerParams(needs_layout_passes=False,
                                             use_tc_tiling_on_sc=False),
        scratch_types=[
            pltpu.VMEM((2, _IDXG, 128), jnp.int32),
            pltpu.VMEM((2, _IDXG, 128), jnp.int32),
            pltpu.VMEM((2, _CHUNK), jnp.float32),
            pltpu.VMEM((2, _CHUNK, d), jnp.float32),
            pltpu.VMEM_SHARED((_N, d), jnp.float32),
            pltpu.SemaphoreType.DMA((2,)),
            pltpu.SemaphoreType.DMA((2,)),
        ],
    )


def _make_spmm2():
    mesh = plsc.VectorSubcoreMesh(core_axis_name="c", subcore_axis_name="s")
    rows_pw = _EPW // 128

    def body(tabf, src2, dst2, wvec, out, tab_v, src_v, dst_v, w_v,
             acc_v, sem):
        c = lax.axis_index("c")
        s = lax.axis_index("s")
        wid = s * _NC + c
        rb = wid * rows_pw

        cps = [pltpu.make_async_copy(tabf, tab_v, sem),
               pltpu.make_async_copy(src2.at[pl.ds(rb, rows_pw)], src_v,
                                     sem),
               pltpu.make_async_copy(dst2.at[pl.ds(rb, rows_pw)], dst_v,
                                     sem),
               pltpu.make_async_copy(wvec.at[pl.ds(wid * _EPW, _EPW)], w_v,
                                     sem)]
        for cp in cps:
            cp.start()

        zero16 = jnp.zeros((16,), jnp.float32)

        @pl.loop(0, 2 * _N // 16)
        def _z(i):
            acc_v[pl.ds(i * 16, 16)] = zero16

        for cp in cps:
            cp.wait()

        @pl.loop(0, _EPW // 16)
        def _grp(g):
            row = g // 8
            off = (g % 8) * 16
            src16 = src_v[row, pl.ds(off, 16)]
            dst16 = dst_v[row, pl.ds(off, 16)]
            w16 = w_v[pl.ds(g * 16, 16)]
            i0 = src16 * 2
            j0 = dst16 * 2
            v0 = plsc.load_gather(tab_v, [i0]) * w16
            v1 = plsc.load_gather(tab_v, [i0 + 1]) * w16
            plsc.addupdate_scatter(acc_v, [j0], v0)
            plsc.addupdate_scatter(acc_v, [j0 + 1], v1)

        pltpu.sync_copy(acc_v, out.at[wid])

    return pl.kernel(
        body,
        out_type=jax.ShapeDtypeStruct((_NW, 2 * _N), jnp.float32),
        mesh=mesh,
        compiler_params=pltpu.CompilerParams(needs_layout_passes=False,
                                             use_tc_tiling_on_sc=False),
        scratch_types=[
            pltpu.VMEM((2 * _N,), jnp.float32),
            pltpu.VMEM((rows_pw, 128), jnp.int32),
            pltpu.VMEM((rows_pw, 128), jnp.int32),
            pltpu.VMEM((_EPW,), jnp.float32),
            pltpu.VMEM((2 * _N,), jnp.float32),
            pltpu.SemaphoreType.DMA,
        ],
    )


def _mid_body(p_ref, w_ref, o_ref):
    h = jnp.maximum(p_ref[0] + p_ref[1], 0.0)
    o_ref[...] = jnp.dot(h, w_ref[...], preferred_element_type=jnp.float32)


def _mid_fuse(p, wfull):
    return pl.pallas_call(
        _mid_body,
        out_shape=jax.ShapeDtypeStruct((_N, 2), jnp.float32),
    )(p, wfull)


def _out_body(p_ref, o_ref):
    t = jnp.sum(p_ref[...], axis=0)
    o_ref[...] = jax.nn.sigmoid(t)


def _out_fuse(p):
    return pl.pallas_call(
        _out_body,
        out_shape=jax.ShapeDtypeStruct((2 * _N,), jnp.float32),
    )(p)


def kernel(features, edge_index, edge_weight, W1, W2, W3, W4):
    src = edge_index[0]
    dst = edge_index[1]

    wc = jnp.zeros((_N, 32), jnp.float32)
    wc = wc.at[:, 0:_H].set(W1)
    wc = wc.at[:, 16:16 + _H].set(W3)
    xw = _big_matmul(features, wc, 400)

    src_p = jnp.zeros((_EPAD,), jnp.int32).at[:_E].set(src)
    src_p = src_p.reshape(_EPAD // 128, 128)
    dst_p = jnp.zeros((_EPAD,), jnp.int32).at[:_E].set(dst)
    dst_p = dst_p.reshape(_EPAD // 128, 128)
    w_p = jnp.zeros((_EPAD,), jnp.float32).at[:_E].set(edge_weight)

    spmm1 = _make_spmm(32, tuple(range(_H)) + tuple(range(16, 16 + _H)))
    z32 = jnp.zeros((_N, 32), jnp.float32)
    p1 = spmm1(xw, src_p, dst_p, w_p, z32)

    wfull = jnp.zeros((32, 2), jnp.float32)
    wfull = wfull.at[0:_H, 0].set(W2[:, 0])
    wfull = wfull.at[16:16 + _H, 1].set(W4[:, 0])
    hw = _mid_fuse(p1, wfull)

    spmm2 = _make_spmm2()
    p2 = spmm2(hw.reshape(-1), src_p, dst_p, w_p)

    o = _out_fuse(p2).reshape(_N, 2)
    return o[:, 0:1], o[:, 1:2]


# speedup vs baseline: 1.2308x; 1.1543x over previous
import jax
import jax.numpy as jnp
from jax import lax
from jax.experimental import pallas as pl
from jax.experimental.pallas import tpu as pltpu
from jax.experimental.pallas import tpu_sc as plsc

_N = 10000
_E = 160000
_H = 10

_NC = 2
_NS = 16
_NW = _NC * _NS
_CHUNK = 640
_IDXG = _CHUNK // 128
_KCH = 8
_EPW = _CHUNK * _KCH
_EPAD = _EPW * _NW
_RPS = 624
_RTAIL = _N - _NS * _RPS


def _mm_body(x_ref, w_ref, oa_ref, ob_ref):
    r = jnp.dot(x_ref[...], w_ref[...], preferred_element_type=jnp.float32)
    oa_ref[...] = r[:, :16]
    ob_ref[...] = r[:, 16:]


def _big_matmul(x, w, bm):
    m, k = x.shape
    return pl.pallas_call(
        _mm_body,
        grid=(m // bm,),
        in_specs=[pl.BlockSpec((bm, k), lambda i: (i, 0)),
                  pl.BlockSpec((k, 32), lambda i: (0, 0))],
        out_specs=[pl.BlockSpec((bm, 16), lambda i: (i, 0)),
                   pl.BlockSpec((bm, 16), lambda i: (i, 0))],
        out_shape=(jax.ShapeDtypeStruct((m, 16), jnp.float32),
                   jax.ShapeDtypeStruct((m, 16), jnp.float32)),
    )(x, w)


_SCOLS = 16
_TCOLS = 4


def _make_spmm1():
    mesh = plsc.VectorSubcoreMesh(core_axis_name="c", subcore_axis_name="s")

    def body(tab_a, tab_b, src2, dst2, wvec, zrows, out, tout,
             src_v, dst_v, w_v, rows_a, rows_b, tacc, acc, gsems, ssems):
        c = lax.axis_index("c")
        s = lax.axis_index("s")
        wid = s * _NC + c
        r0 = s * _RPS

        pltpu.sync_copy(zrows.at[pl.ds(r0, _RPS)], acc.at[pl.ds(r0, _RPS)])

        @pl.when(s == 0)
        def _ztail():
            pltpu.sync_copy(zrows.at[pl.ds(_NS * _RPS, _RTAIL)],
                            acc.at[pl.ds(_NS * _RPS, _RTAIL)])

        zero16 = jnp.zeros((16,), jnp.float32)

        @pl.loop(0, _N // 16, unroll=4)
        def _zt(i):
            for r in range(_TCOLS):
                tacc[r, pl.ds(i * 16, 16)] = zero16

        plsc.subcore_barrier()

        lanes = lax.iota(jnp.int32, 16)
        quad = lanes // 4
        cmod = lanes % 4

        def fetch(k, b):
            base = wid * _EPW + k * _CHUNK
            rb = wid * (_EPW // 128) + k * _IDXG
            pltpu.sync_copy(src2.at[pl.ds(rb, _IDXG)], src_v.at[b])
            pltpu.sync_copy(dst2.at[pl.ds(rb, _IDXG)], dst_v.at[b])
            pltpu.sync_copy(wvec.at[pl.ds(base, _CHUNK)], w_v.at[b])
            for j in range(_IDXG):
                pltpu.async_copy(tab_a.at[src_v.at[b, j]],
                                 rows_a.at[b, pl.ds(j * 128, 128)],
                                 gsems.at[b])
                pltpu.async_copy(tab_b.at[src_v.at[b, j]],
                                 rows_b.at[b, pl.ds(j * 128, 128)],
                                 gsems.at[b])

        def scatter_descs(b):
            return [pltpu.make_async_copy(rows_a.at[b, pl.ds(j * 128, 128)],
                                          acc.at[dst_v.at[b, j]],
                                          ssems.at[b])
                    for j in range(_IDXG)]

        def process(k, b):
            for j in range(_IDXG):
                pltpu.make_async_copy(tab_a.at[src_v.at[b, j]],
                                      rows_a.at[b, pl.ds(j * 128, 128)],
                                      gsems.at[b]).wait()
                pltpu.make_async_copy(tab_b.at[src_v.at[b, j]],
                                      rows_b.at[b, pl.ds(j * 128, 128)],
                                      gsems.at[b]).wait()

                @pl.loop(0, 128 // 16, unroll=2)
                def _grp(g0, j=j):
                    g = j * 8 + g0
                    w16 = w_v[b, pl.ds(g * 16, 16)]
                    rid = g * 16 + lanes
                    for dcol in range(_SCOLS):
                        cidx = jnp.full((16,), dcol, jnp.int32)
                        v = plsc.load_gather(rows_a.at[b], [rid, cidx])
                        plsc.store_scatter(rows_a.at[b], [rid, cidx],
                                           v * w16)
                    for q in range(4):
                        erid = g * 16 + q * 4 + quad
                        lidx = g0 * 16 + q * 4 + quad
                        vals = plsc.load_gather(rows_b.at[b],
                                                [erid, cmod])
                        wv = plsc.load_gather(w_v.at[b], [erid])
                        dv = plsc.load_gather(dst_v.at[b, j], [lidx])
                        plsc.addupdate_scatter(tacc, [cmod, dv], vals * wv)

                pltpu.make_async_copy(rows_a.at[b, pl.ds(j * 128, 128)],
                                      acc.at[dst_v.at[b, j]],
                                      ssems.at[b]).start(add=True)

        fetch(0, 0)
        fetch(1, 1)

        @pl.loop(0, _KCH // 2)
        def _pair(i):
            k = i * 2
            process(k, 0)
            process(k + 1, 1)

            @pl.when(k + 2 < _KCH)
            def _fetch_a():
                for dsc in scatter_descs(0):
                    dsc.wait()
                fetch(k + 2, 0)

            @pl.when(k + 3 < _KCH)
            def _fetch_b():
                for dsc in scatter_descs(1):
                    dsc.wait()
                fetch(k + 3, 1)

        for b in (0, 1):
            for dsc in scatter_descs(b):
                dsc.wait()

        pltpu.sync_copy(tacc, tout.at[wid])

        plsc.subcore_barrier()
        pltpu.sync_copy(acc.at[pl.ds(r0, _RPS)],
                        out.at[c, pl.ds(r0, _RPS)])

        @pl.when(s == 0)
        def _otail():
            pltpu.sync_copy(acc.at[pl.ds(_NS * _RPS, _RTAIL)],
                            out.at[c, pl.ds(_NS * _RPS, _RTAIL)])

    return pl.kernel(
        body,
        out_type=(jax.ShapeDtypeStruct((_NC, _N, _SCOLS), jnp.float32),
                  jax.ShapeDtypeStruct((_NW, _TCOLS, _N), jnp.float32)),
        mesh=mesh,
        compiler_params=pltpu.CompilerParams(needs_layout_passes=False,
                                             use_tc_tiling_on_sc=False),
        scratch_types=[
            pltpu.VMEM((2, _IDXG, 128), jnp.int32),
            pltpu.VMEM((2, _IDXG, 128), jnp.int32),
            pltpu.VMEM((2, _CHUNK), jnp.float32),
            pltpu.VMEM((2, _CHUNK, 16), jnp.float32),
            pltpu.VMEM((2, _CHUNK, 16), jnp.float32),
            pltpu.VMEM((_TCOLS, _N), jnp.float32),
            pltpu.VMEM_SHARED((_N, _SCOLS), jnp.float32),
            pltpu.SemaphoreType.DMA((2,)),
            pltpu.SemaphoreType.DMA((2,)),
        ],
    )


def _make_spmm2():
    mesh = plsc.VectorSubcoreMesh(core_axis_name="c", subcore_axis_name="s")
    rows_pw = _EPW // 128

    def body(tabf, src2, dst2, wvec, out, tab_v, src_v, dst_v, w_v,
             acc_v, sem):
        c = lax.axis_index("c")
        s = lax.axis_index("s")
        wid = s * _NC + c
        rb = wid * rows_pw

        cps = [pltpu.make_async_copy(tabf, tab_v, sem),
               pltpu.make_async_copy(src2.at[pl.ds(rb, rows_pw)], src_v,
                                     sem),
               pltpu.make_async_copy(dst2.at[pl.ds(rb, rows_pw)], dst_v,
                                     sem),
               pltpu.make_async_copy(wvec.at[pl.ds(wid * _EPW, _EPW)], w_v,
                                     sem)]
        for cp in cps:
            cp.start()

        zero16 = jnp.zeros((16,), jnp.float32)

        @pl.loop(0, 2 * _N // 16)
        def _z(i):
            acc_v[pl.ds(i * 16, 16)] = zero16

        for cp in cps:
            cp.wait()

        @pl.loop(0, _EPW // 16)
        def _grp(g):
            row = g // 8
            off = (g % 8) * 16
            src16 = src_v[row, pl.ds(off, 16)]
            dst16 = dst_v[row, pl.ds(off, 16)]
            w16 = w_v[pl.ds(g * 16, 16)]
            i0 = src16 * 2
            j0 = dst16 * 2
            v0 = plsc.load_gather(tab_v, [i0]) * w16
            v1 = plsc.load_gather(tab_v, [i0 + 1]) * w16
            plsc.addupdate_scatter(acc_v, [j0], v0)
            plsc.addupdate_scatter(acc_v, [j0 + 1], v1)

        pltpu.sync_copy(acc_v, out.at[wid])

    return pl.kernel(
        body,
        out_type=jax.ShapeDtypeStruct((_NW, 2 * _N), jnp.float32),
        mesh=mesh,
        compiler_params=pltpu.CompilerParams(needs_layout_passes=False,
                                             use_tc_tiling_on_sc=False),
        scratch_types=[
            pltpu.VMEM((2 * _N,), jnp.float32),
            pltpu.VMEM((rows_pw, 128), jnp.int32),
            pltpu.VMEM((rows_pw, 128), jnp.int32),
            pltpu.VMEM((_EPW,), jnp.float32),
            pltpu.VMEM((2 * _N,), jnp.float32),
            pltpu.SemaphoreType.DMA,
        ],
    )


def _mid_body(p_ref, t_ref, wa_ref, wb_ref, o_ref):
    h1 = jnp.maximum(p_ref[0] + p_ref[1], 0.0)
    t = jnp.sum(t_ref[...], axis=0)
    h2 = jnp.maximum(t, 0.0)
    c1 = jnp.sum(h2 * wb_ref[...], axis=0)
    sel = lax.broadcasted_iota(jnp.int32, (1, 2), 1).astype(jnp.float32)
    o_ref[...] = (jnp.dot(h1, wa_ref[...],
                          preferred_element_type=jnp.float32)
                  + c1[:, None] * sel)


def _mid_fuse(p, t, wa, wb):
    return pl.pallas_call(
        _mid_body,
        out_shape=jax.ShapeDtypeStruct((_N, 2), jnp.float32),
    )(p, t, wa, wb)


def _out_body(p_ref, o_ref):
    t = jnp.sum(p_ref[...], axis=0)
    o_ref[...] = jax.nn.sigmoid(t)


def _out_fuse(p):
    return pl.pallas_call(
        _out_body,
        out_shape=jax.ShapeDtypeStruct((2 * _N,), jnp.float32),
    )(p)


def kernel(features, edge_index, edge_weight, W1, W2, W3, W4):
    src = edge_index[0]
    dst = edge_index[1]

    wc = jnp.zeros((_N, 32), jnp.float32)
    wc = wc.at[:, 0:_H].set(W1)
    wc = wc.at[:, _H:2 * _H].set(W3)
    xw_a, xw_b = _big_matmul(features, wc, 400)

    src_p = jnp.zeros((_EPAD,), jnp.int32).at[:_E].set(src)
    src_p = src_p.reshape(_EPAD // 128, 128)
    dst_p = jnp.zeros((_EPAD,), jnp.int32).at[:_E].set(dst)
    dst_p = dst_p.reshape(_EPAD // 128, 128)
    w_p = jnp.zeros((_EPAD,), jnp.float32).at[:_E].set(edge_weight)

    z16 = jnp.zeros((_N, _SCOLS), jnp.float32)
    p1, t1 = _make_spmm1()(xw_a, xw_b, src_p, dst_p, w_p, z16)

    wa = jnp.zeros((_SCOLS, 2), jnp.float32)
    wa = wa.at[0:_H, 0].set(W2[:, 0])
    wa = wa.at[_H:_SCOLS, 1].set(W4[0:_SCOLS - _H, 0])
    wb = W4[_SCOLS - _H:_H, 0:1]
    hw = _mid_fuse(p1, t1, wa, wb)

    spmm2 = _make_spmm2()
    p2 = spmm2(hw.reshape(-1), src_p, dst_p, w_p)

    o = _out_fuse(p2).reshape(_N, 2)
    return o[:, 0:1], o[:, 1:2]
